# baseline (speedup 1.0000x reference)
import functools
import math

import jax
import jax.numpy as jnp
from jax import lax
from jax.experimental import pallas as pl
from jax.experimental.pallas import tpu as pltpu
from jax.experimental.pallas import tpu_sc as plsc

D_MODEL = 64
SCALE = math.sqrt(D_MODEL)
VOCAB = 1_000_000
PAIRS = VOCAB // 2

NUM_CORES = 2
NUM_SUBCORES = 16
NUM_WORKERS = NUM_CORES * NUM_SUBCORES
LANES = 16

P1_VBLK = 256
HALF = 500224
P1_GRID = HALF // P1_VBLK
N_VBLKS = -(-VOCAB // P1_VBLK)

SEQ = 200
BATCH = 4096
BBLK = BATCH // NUM_WORKERS


def _p1_body(a1_ref, a2_ref, out_ref):
    r = lax.broadcasted_iota(jnp.int32, (D_MODEL, D_MODEL), 0)
    c = lax.broadcasted_iota(jnp.int32, (D_MODEL, D_MODEL), 1)
    ident = (r == c).astype(jnp.float32)

    def tr(a):
        return lax.dot_general(
            a, ident, (((0,), (0,)), ((), ())),
            preferred_element_type=jnp.float32,
        )

    out_ref[:, 0:D_MODEL] = tr(a1_ref[...]) * SCALE
    out_ref[:, D_MODEL : 2 * D_MODEL] = tr(a2_ref[...]) * SCALE


@functools.cache
def _p1_call():
    return pl.pallas_call(
        _p1_body,
        grid=(P1_GRID,),
        in_specs=[
            pl.BlockSpec((D_MODEL, P1_VBLK), lambda i: (0, i)),
            pl.BlockSpec(
                (D_MODEL, P1_VBLK),
                lambda i: (0, jnp.minimum(i + P1_GRID, N_VBLKS - 1)),
            ),
        ],
        out_specs=pl.BlockSpec((P1_VBLK, 2 * D_MODEL), lambda i: (i, 0)),
        out_shape=jax.ShapeDtypeStruct((HALF, 2 * D_MODEL), jnp.float32),
    )


@functools.cache
def _p2_kernel():
    mesh = plsc.VectorSubcoreMesh(core_axis_name="c", subcore_axis_name="s")

    @functools.partial(
        pl.kernel,
        out_type=jax.ShapeDtypeStruct(
            (SEQ, D_MODEL // 8, NUM_WORKERS, 8, BBLK), jnp.float32
        ),
        mesh=mesh,
        scratch_types=[
            pltpu.VMEM((SEQ, BBLK), jnp.int32),
            pltpu.VMEM((SEQ, BBLK), jnp.int32),
            pltpu.VMEM((BBLK, 2 * D_MODEL), jnp.float32),
            pltpu.VMEM((BBLK, 2 * D_MODEL), jnp.float32),
            pltpu.VMEM((D_MODEL // 8, 8, BBLK), jnp.float32),
            pltpu.VMEM((D_MODEL // 8, 8, BBLK), jnp.float32),
            pltpu.SemaphoreType.DMA,
            pltpu.SemaphoreType.DMA,
            pltpu.SemaphoreType.DMA,
            pltpu.SemaphoreType.DMA,
        ],
        compiler_params=pltpu.CompilerParams(needs_layout_passes=False),
    )
    def body(xt_hbm, tab2_hbm, out_hbm, idx_raw, p_buf, pr0, pr1, ob0, ob1,
             gp0, gp1, so0, so1):
        prs = (pr0, pr1)
        obs = (ob0, ob1)
        gsem = (gp0, gp1)
        ssem = (so0, so1)
        wid = lax.axis_index("s") * NUM_CORES + lax.axis_index("c")
        b0 = wid * BBLK

        pltpu.sync_copy(xt_hbm.at[:, pl.ds(b0, BBLK)], idx_raw)

        @pl.loop(0, SEQ)
        def _(j):
            for t in range(BBLK // LANES):
                sl = pl.ds(t * LANES, LANES)
                v = idx_raw[j, sl]
                p_buf[j, sl] = jnp.where(v >= HALF, v - HALF, v)

        def issue_gather(j, b):
            pltpu.async_copy(tab2_hbm.at[p_buf.at[j]], prs[b], gsem[b])

        def wait_gather(j, b):
            pltpu.make_async_copy(tab2_hbm.at[p_buf.at[j]], prs[b], gsem[b]).wait()

        def issue_out(j, b):
            pltpu.async_copy(obs[b], out_hbm.at[j, :, wid, :, :], ssem[b])

        def wait_out(b):
            pltpu.make_async_copy(
                obs[b], out_hbm.at[0, :, wid, :, :], ssem[b]
            ).wait()

        it = lax.broadcasted_iota(jnp.int32, (16,), 0)

        def select_transpose(j, b):
            for j2 in range(BBLK // LANES):
                rows = j2 * LANES + it
                vj = idx_raw[j, pl.ds(j2 * LANES, LANES)]
                hcol = jnp.where(vj >= HALF, D_MODEL, 0)

                @pl.loop(0, D_MODEL)
                def _(d):
                    v = plsc.load_gather(prs[b], [rows, hcol + d])
                    obs[b][d // 8, d % 8, pl.ds(j2 * LANES, LANES)] = v

        issue_gather(0, 0)

        @pl.loop(0, SEQ, step=2)
        def outer(j0):
            for b in range(2):
                j = j0 + b

                @pl.when(j + 1 < SEQ)
                def _():
                    issue_gather(j + 1, 1 - b)

                wait_gather(j, b)

                @pl.when(j >= 2)
                def _():
                    wait_out(b)

                select_transpose(j, b)
                issue_out(j, b)

        for b in range(2):
            wait_out(b)

    return body


def kernel(x, table):
    tabt = table.T
    tab2 = _p1_call()(tabt, tabt)
    xt = x.astype(jnp.int32).T
    out5 = _p2_kernel()(xt, tab2)
    out = out5.transpose(2, 4, 0, 1, 3).reshape(BATCH, SEQ, D_MODEL)
    return out


# speedup vs baseline: 1.3057x; 1.3057x over previous
import functools
import math

import jax
import jax.numpy as jnp
from jax import lax
from jax.experimental import pallas as pl
from jax.experimental.pallas import tpu as pltpu
from jax.experimental.pallas import tpu_sc as plsc

D_MODEL = 64
SCALE = math.sqrt(D_MODEL)
VOCAB = 1_000_000
PAIRS = VOCAB // 2

NUM_CORES = 2
NUM_SUBCORES = 16
NUM_WORKERS = NUM_CORES * NUM_SUBCORES
LANES = 16

P1_VBLK = 512
HALF = 500224
P1_GRID = HALF // P1_VBLK
N_VBLKS = -(-VOCAB // P1_VBLK)

SEQ = 200
BATCH = 4096
BBLK = BATCH // NUM_WORKERS


def _p1_body(a1_ref, a2_ref, out_ref):
    out_ref[:, 0:D_MODEL] = a1_ref[...].T * SCALE
    out_ref[:, D_MODEL : 2 * D_MODEL] = a2_ref[...].T * SCALE


@functools.cache
def _p1_call():
    return pl.pallas_call(
        _p1_body,
        grid=(P1_GRID,),
        in_specs=[
            pl.BlockSpec((D_MODEL, P1_VBLK), lambda i: (0, i)),
            pl.BlockSpec(
                (D_MODEL, P1_VBLK),
                lambda i: (0, jnp.minimum(i + P1_GRID, N_VBLKS - 1)),
            ),
        ],
        out_specs=pl.BlockSpec((P1_VBLK, 2 * D_MODEL), lambda i: (i, 0)),
        out_shape=jax.ShapeDtypeStruct((HALF, 2 * D_MODEL), jnp.float32),
    )


@functools.cache
def _p2_kernel():
    mesh = plsc.VectorSubcoreMesh(core_axis_name="c", subcore_axis_name="s")

    @functools.partial(
        pl.kernel,
        out_type=jax.ShapeDtypeStruct(
            (SEQ, D_MODEL // 8, NUM_WORKERS, 8, BBLK), jnp.float32
        ),
        mesh=mesh,
        scratch_types=[
            pltpu.VMEM((SEQ, BBLK), jnp.int32),
            pltpu.VMEM((SEQ, BBLK), jnp.int32),
            pltpu.VMEM((BBLK, D_MODEL), jnp.float32),
            pltpu.VMEM((BBLK, D_MODEL), jnp.float32),
            pltpu.VMEM((D_MODEL // 8, 8, BBLK), jnp.float32),
            pltpu.VMEM((D_MODEL // 8, 8, BBLK), jnp.float32),
            pltpu.SemaphoreType.DMA,
            pltpu.SemaphoreType.DMA,
            pltpu.SemaphoreType.DMA,
            pltpu.SemaphoreType.DMA,
        ],
        compiler_params=pltpu.CompilerParams(
            needs_layout_passes=False, use_tc_tiling_on_sc=False
        ),
    )
    def body(xt_hbm, tab2_hbm, out_hbm, idx_raw, p_buf, pr0, pr1, ob0, ob1,
             gp0, gp1, so0, so1):
        prs = (pr0, pr1)
        obs = (ob0, ob1)
        gsem = (gp0, gp1)
        ssem = (so0, so1)
        wid = lax.axis_index("s") * NUM_CORES + lax.axis_index("c")
        b0 = wid * BBLK

        pltpu.sync_copy(xt_hbm.at[:, pl.ds(b0, BBLK)], idx_raw)

        @pl.loop(0, SEQ)
        def _(j):
            for t in range(BBLK // LANES):
                sl = pl.ds(t * LANES, LANES)
                v = idx_raw[j, sl]
                p_buf[j, sl] = jnp.where(
                    v >= HALF, 2 * (v - HALF) + 1, 2 * v
                )

        def issue_gather(j, b):
            pltpu.async_copy(tab2_hbm.at[p_buf.at[j]], prs[b], gsem[b])

        def wait_gather(j, b):
            pltpu.make_async_copy(tab2_hbm.at[p_buf.at[j]], prs[b], gsem[b]).wait()

        def issue_out(j, b):
            pltpu.async_copy(obs[b], out_hbm.at[j, :, wid, :, :], ssem[b])

        def wait_out(b):
            pltpu.make_async_copy(
                obs[b], out_hbm.at[0, :, wid, :, :], ssem[b]
            ).wait()

        it = lax.broadcasted_iota(jnp.int32, (16,), 0)

        zero16 = it * 0

        def select_transpose(j, b):
            for j2 in range(BBLK // LANES):
                rows = j2 * LANES + it

                @pl.loop(0, D_MODEL, unroll=8)
                def _(d):
                    v = plsc.load_gather(prs[b], [rows, zero16 + d])
                    obs[b][d // 8, d % 8, pl.ds(j2 * LANES, LANES)] = v

        issue_gather(0, 0)

        @pl.loop(0, SEQ, step=2)
        def outer(j0):
            for b in range(2):
                j = j0 + b

                @pl.when(j + 1 < SEQ)
                def _():
                    issue_gather(j + 1, 1 - b)

                wait_gather(j, b)

                @pl.when(j >= 2)
                def _():
                    wait_out(b)

                select_transpose(j, b)
                issue_out(j, b)

        for b in range(2):
            wait_out(b)

    return body


def kernel(x, table):
    tabt = table.T
    tab2 = _p1_call()(tabt, tabt)
    tab64 = tab2.reshape(2 * HALF, D_MODEL)
    xt = x.astype(jnp.int32).T
    out5 = _p2_kernel()(xt, tab64)
    out = out5.transpose(2, 4, 0, 1, 3).reshape(BATCH, SEQ, D_MODEL)
    return out


# speedup vs baseline: 2.0323x; 1.5565x over previous
import functools
import math

import jax
import jax.numpy as jnp
from jax import lax
from jax.experimental import pallas as pl
from jax.experimental.pallas import tpu as pltpu
from jax.experimental.pallas import tpu_sc as plsc

D_MODEL = 64
SCALE = math.sqrt(D_MODEL)

NUM_CORES = 2
NUM_SUBCORES = 16
NUM_WORKERS = NUM_CORES * NUM_SUBCORES
LANES = 16

CHUNK = 128
K = 2
NBUF = 4
DEPTH = 2
STEP_ROWS = K * CHUNK


@functools.cache
def _emb_kernel(B: int):
    b_per_w = B // NUM_WORKERS
    n_gathers = b_per_w // CHUNK
    n_steps = b_per_w // STEP_ROWS
    assert b_per_w % STEP_ROWS == 0 and n_steps % NBUF == 0
    mesh = plsc.VectorSubcoreMesh(core_axis_name="c", subcore_axis_name="s")

    @functools.partial(
        pl.kernel,
        out_type=jax.ShapeDtypeStruct((B, D_MODEL), jnp.float32),
        mesh=mesh,
        scratch_types=[
            pltpu.VMEM((n_gathers, CHUNK), jnp.int32),
        ]
        + [pltpu.VMEM((STEP_ROWS, D_MODEL), jnp.float32) for _ in range(NBUF)]
        + [pltpu.SemaphoreType.DMA for _ in range(2 * NBUF)],
        compiler_params=pltpu.CompilerParams(use_tc_tiling_on_sc=False),
    )
    def body(idx_hbm, table_hbm, out_hbm, idx_all, *bufs_and_sems):
        rows = bufs_and_sems[:NBUF]
        sem_g = bufs_and_sems[NBUF : 2 * NBUF]
        sem_s = bufs_and_sems[2 * NBUF : 3 * NBUF]

        wid = lax.axis_index("s") * NUM_CORES + lax.axis_index("c")
        wbase = wid * b_per_w

        def issue_gather(s, b):
            for k in range(K):
                j = s * K + k
                pltpu.async_copy(
                    table_hbm.at[idx_all.at[j]],
                    rows[b].at[pl.ds(k * CHUNK, CHUNK)],
                    sem_g[b],
                )

        def wait_gather(s, b):
            for k in range(K):
                j = s * K + k
                pltpu.make_async_copy(
                    table_hbm.at[idx_all.at[j]],
                    rows[b].at[pl.ds(k * CHUNK, CHUNK)],
                    sem_g[b],
                ).wait()

        def issue_scatter(s, b):
            obase = wbase + s * STEP_ROWS
            pltpu.async_copy(rows[b], out_hbm.at[pl.ds(obase, STEP_ROWS)], sem_s[b])

        def wait_scatter(b):
            pltpu.make_async_copy(
                rows[b], out_hbm.at[pl.ds(wbase, STEP_ROWS)], sem_s[b]
            ).wait()

        pltpu.sync_copy(idx_hbm.at[wid], idx_all)

        for s0 in range(DEPTH):
            issue_gather(s0, s0 % NBUF)

        @pl.loop(0, n_steps, step=NBUF)
        def outer(g):
            for b in range(NBUF):
                s = g + b
                wait_gather(s, b)

                @pl.loop(0, STEP_ROWS, unroll=4)
                def scale_row(i):
                    for j in range(D_MODEL // LANES):
                        sl = pl.ds(j * LANES, LANES)
                        rows[b][i, sl] = rows[b][i, sl] * SCALE

                issue_scatter(s, b)

                bn = (b + DEPTH) % NBUF

                @pl.when(s + DEPTH < n_steps)
                def _():
                    @pl.when(s + DEPTH >= NBUF)
                    def _():
                        wait_scatter(bn)

                    issue_gather(s + DEPTH, bn)

        for b in range(NBUF):
            wait_scatter(b)

    return body


def kernel(x, table):
    B = x.size
    b_per_w = B // NUM_WORKERS
    idx3 = x.reshape(NUM_WORKERS, b_per_w // CHUNK, CHUNK).astype(jnp.int32)
    out = _emb_kernel(B)(idx3, table)
    return out.reshape(x.shape + (D_MODEL,))
